# baseline (speedup 1.0000x reference)
import functools

import jax
import jax.numpy as jnp
from jax import lax
from jax.experimental import pallas as pl
from jax.experimental.pallas import tpu as pltpu
from jax.experimental.pallas import tpu_sc as plsc

_M = 256
_D = 256
_N = _M * _M

_NC = 2
_NS = 16
_L = 16
_NW = _NC * _NS
_K = 1
_NK = _N // _K
_RPW = _NK // _NW
_CH = 64
_CPA = _M // _CH


def _scalar_stage(x_ref, xT_ref, mem_ref, memT_ref, t_ref):
    x = x_ref[...]
    xT = xT_ref[...]
    mem = mem_ref[...]
    memT = memT_ref[...]
    f32 = jnp.float32
    rx = 1.0 / jnp.maximum(jnp.sqrt(jnp.sum(x * x, axis=1, keepdims=True)), 1e-8)
    xn_row = x * rx
    xn_col = xT * rx
    q1_row = jnp.sum(memT, axis=0, keepdims=True)
    q2_row = jnp.sum(memT * memT, axis=0, keepdims=True)
    q2_col = jnp.sum(mem * mem, axis=1, keepdims=True)
    rn_row = 1.0 / jnp.maximum(jnp.sqrt(q2_row), 1e-8)
    rn_col = 1.0 / jnp.maximum(jnp.sqrt(q2_col), 1e-8)
    mx_col = jnp.dot(mem, xn_col, preferred_element_type=f32)
    mx_row = jnp.dot(xn_row, memT, preferred_element_type=f32)
    sims_col = mx_col * rn_col
    sims_row = mx_row * rn_row
    act_col = (sims_col > 0.3).astype(f32)
    act_row = (sims_row > 0.3).astype(f32)
    raw = jnp.dot(mem, memT, preferred_element_type=f32)
    s = act_col * (raw * rn_col * rn_row) + sims_row
    v = xn_row + jnp.dot(act_row * rn_row, mem,
                         preferred_element_type=f32) * (1.0 / _M)
    sv = jnp.sum(v, axis=1, keepdims=True)
    dv_row = jnp.dot(v, memT, preferred_element_type=f32)
    den = jnp.maximum(jnp.sqrt(q2_row + 2.0 * s * q1_row + float(_D) * s * s),
                      1e-8)
    t_ref[...] = s + (dv_row + s * sv) / den


def _sc_assemble(noise_hbm, mem_hbm, tE_hbm, out_hbm,
                 nbuf0, nbuf1, obuf0, obuf1, mbuf0, mbuf1, tbuf0, tbuf1,
                 sin0, sin1, sout0, sout1):
    wid = lax.axis_index("s") * _NC + lax.axis_index("c")
    row0 = wid * _RPW
    nchunks = _RPW // _CH

    def start_in(c, nb, mb, tb, sem):
        pltpu.async_copy(noise_hbm.at[pl.ds(row0 + c * _CH, _CH)], nb, sem)
        pltpu.async_copy(tE_hbm.at[pl.ds(row0 + c * _CH, _CH)], tb, sem)
        pltpu.async_copy(mem_hbm.at[pl.ds((c % _CPA) * _CH, _CH)], mb, sem)

    def wait_in(nb, mb, tb, sem):
        pltpu.make_async_copy(noise_hbm.at[pl.ds(row0, _CH)], nb, sem).wait()
        pltpu.make_async_copy(tE_hbm.at[pl.ds(row0, _CH)], tb, sem).wait()
        pltpu.make_async_copy(mem_hbm.at[pl.ds(0, _CH)], mb, sem).wait()

    def start_out(c, ob, sem):
        pltpu.async_copy(ob, out_hbm.at[pl.ds(row0 + c * _CH, _CH)], sem)

    def wait_out(ob, sem):
        pltpu.make_async_copy(ob, out_hbm.at[pl.ds(row0, _CH)], sem).wait()

    def compute(c, nb, mb, tb, ob):
        @plsc.parallel_loop(0, _CH, unroll=8)
        def _row(rr):
            tvec = tb[rr, pl.ds(0, _L)]
            for db in range(_D // _L):
                ob[rr, pl.ds(db * _L, _L)] = (
                    nb[rr, pl.ds(db * _L, _L)]
                    + mb[rr, pl.ds(db * _L, _L)]
                    + tvec)

    start_in(0, nbuf0, mbuf0, tbuf0, sin0)
    start_in(1, nbuf1, mbuf1, tbuf1, sin1)
    wait_in(nbuf0, mbuf0, tbuf0, sin0)
    compute(0, nbuf0, mbuf0, tbuf0, obuf0)
    start_out(0, obuf0, sout0)
    start_in(2, nbuf0, mbuf0, tbuf0, sin0)
    wait_in(nbuf1, mbuf1, tbuf1, sin1)
    compute(1, nbuf1, mbuf1, tbuf1, obuf1)
    start_out(1, obuf1, sout1)
    start_in(3, nbuf1, mbuf1, tbuf1, sin1)

    def pair_body(cc, carry):
        c0 = cc * 2
        wait_in(nbuf0, mbuf0, tbuf0, sin0)
        wait_out(obuf0, sout0)
        compute(c0, nbuf0, mbuf0, tbuf0, obuf0)
        start_out(c0, obuf0, sout0)
        start_in(c0 + 2, nbuf0, mbuf0, tbuf0, sin0)
        wait_in(nbuf1, mbuf1, tbuf1, sin1)
        wait_out(obuf1, sout1)
        compute(c0 + 1, nbuf1, mbuf1, tbuf1, obuf1)
        start_out(c0 + 1, obuf1, sout1)
        start_in(c0 + 3, nbuf1, mbuf1, tbuf1, sin1)
        return carry

    lax.fori_loop(1, nchunks // 2 - 1, pair_body, 0)

    c0 = nchunks - 2
    wait_in(nbuf0, mbuf0, tbuf0, sin0)
    wait_out(obuf0, sout0)
    compute(c0, nbuf0, mbuf0, tbuf0, obuf0)
    start_out(c0, obuf0, sout0)
    wait_in(nbuf1, mbuf1, tbuf1, sin1)
    wait_out(obuf1, sout1)
    compute(c0 + 1, nbuf1, mbuf1, tbuf1, obuf1)
    start_out(c0 + 1, obuf1, sout1)
    wait_out(obuf0, sout0)
    wait_out(obuf1, sout1)


def kernel(x, mem):
    noise = jax.random.normal(jax.random.key(42), (_N, _D), jnp.float32) * 0.1
    t = pl.pallas_call(
        _scalar_stage,
        out_shape=jax.ShapeDtypeStruct((_M, _M), jnp.float32),
    )(x, x.T, mem, mem.T)
    tE = jnp.broadcast_to(t.reshape(_N, 1), (_N, _L))
    mesh = plsc.VectorSubcoreMesh(core_axis_name="c", subcore_axis_name="s")
    assemble = functools.partial(
        pl.kernel,
        mesh=mesh,
        out_type=jax.ShapeDtypeStruct((_NK, _D), jnp.float32),
        scratch_types=[
            pltpu.VMEM((_CH, _D), jnp.float32),
            pltpu.VMEM((_CH, _D), jnp.float32),
            pltpu.VMEM((_CH, _D), jnp.float32),
            pltpu.VMEM((_CH, _D), jnp.float32),
            pltpu.VMEM((_CH, _D), jnp.float32),
            pltpu.VMEM((_CH, _D), jnp.float32),
            pltpu.VMEM((_CH, _L), jnp.float32),
            pltpu.VMEM((_CH, _L), jnp.float32),
            pltpu.SemaphoreType.DMA,
            pltpu.SemaphoreType.DMA,
            pltpu.SemaphoreType.DMA,
            pltpu.SemaphoreType.DMA,
        ],
        compiler_params=pltpu.CompilerParams(use_tc_tiling_on_sc=True),
    )(_sc_assemble)
    parts = []
    for k in range(_K):
        nz_k = lax.slice(noise, (k * _NK, 0), ((k + 1) * _NK, _D))
        tE_k = lax.slice(tE, (k * _NK, 0), ((k + 1) * _NK, _L))
        parts.append(assemble(nz_k, mem, tE_k))
    return jnp.concatenate(parts, axis=0)


# speedup vs baseline: 1.0872x; 1.0872x over previous
import functools

import jax
import jax.numpy as jnp
from jax import lax
from jax.experimental import pallas as pl
from jax.experimental.pallas import tpu as pltpu
from jax.experimental.pallas import tpu_sc as plsc

_M = 256
_D = 256
_N = _M * _M

_NC = 2
_NS = 16
_L = 16
_NW = _NC * _NS
_K = 1
_NK = _N // _K
_RPW = _NK // _NW
_CH = 32
_CPA = _M // _CH


def _scalar_stage(x_ref, xT_ref, mem_ref, memT_ref, t_ref):
    x = x_ref[...]
    xT = xT_ref[...]
    mem = mem_ref[...]
    memT = memT_ref[...]
    f32 = jnp.float32
    rx = 1.0 / jnp.maximum(jnp.sqrt(jnp.sum(x * x, axis=1, keepdims=True)), 1e-8)
    xn_row = x * rx
    xn_col = xT * rx
    q1_row = jnp.sum(memT, axis=0, keepdims=True)
    q2_row = jnp.sum(memT * memT, axis=0, keepdims=True)
    q2_col = jnp.sum(mem * mem, axis=1, keepdims=True)
    rn_row = 1.0 / jnp.maximum(jnp.sqrt(q2_row), 1e-8)
    rn_col = 1.0 / jnp.maximum(jnp.sqrt(q2_col), 1e-8)
    mx_col = jnp.dot(mem, xn_col, preferred_element_type=f32)
    mx_row = jnp.dot(xn_row, memT, preferred_element_type=f32)
    sims_col = mx_col * rn_col
    sims_row = mx_row * rn_row
    act_col = (sims_col > 0.3).astype(f32)
    act_row = (sims_row > 0.3).astype(f32)
    raw = jnp.dot(mem, memT, preferred_element_type=f32)
    s = act_col * (raw * rn_col * rn_row) + sims_row
    v = xn_row + jnp.dot(act_row * rn_row, mem,
                         preferred_element_type=f32) * (1.0 / _M)
    sv = jnp.sum(v, axis=1, keepdims=True)
    dv_row = jnp.dot(v, memT, preferred_element_type=f32)
    den = jnp.maximum(jnp.sqrt(q2_row + 2.0 * s * q1_row + float(_D) * s * s),
                      1e-8)
    t_ref[...] = s + (dv_row + s * sv) / den


def _sc_assemble(noise_hbm, mem_hbm, tE_hbm, out_hbm, mem_v,
                 nbuf0, nbuf1, obuf0, obuf1, tbuf0, tbuf1,
                 sin0, sin1, sout0, sout1):
    wid = lax.axis_index("s") * _NC + lax.axis_index("c")
    pltpu.sync_copy(mem_hbm, mem_v)
    row0 = wid * _RPW
    nchunks = _RPW // _CH

    def start_in(c, nb, tb, sem):
        pltpu.async_copy(noise_hbm.at[pl.ds(row0 + c * _CH, _CH)], nb, sem)
        pltpu.async_copy(tE_hbm.at[pl.ds(row0 + c * _CH, _CH)], tb, sem)

    def wait_in(nb, tb, sem):
        pltpu.make_async_copy(noise_hbm.at[pl.ds(row0, _CH)], nb, sem).wait()
        pltpu.make_async_copy(tE_hbm.at[pl.ds(row0, _CH)], tb, sem).wait()

    def start_out(c, ob, sem):
        pltpu.async_copy(ob, out_hbm.at[pl.ds(row0 + c * _CH, _CH)], sem)

    def wait_out(ob, sem):
        pltpu.make_async_copy(ob, out_hbm.at[pl.ds(row0, _CH)], sem).wait()

    def compute(c, nb, tb, ob):
        i0 = (c % _CPA) * _CH

        @plsc.parallel_loop(0, _CH, unroll=8)
        def _row(rr):
            tvec = tb[rr, pl.ds(0, _L)]
            i = i0 + rr
            for db in range(_D // _L):
                ob[rr, pl.ds(db * _L, _L)] = (
                    nb[rr, pl.ds(db * _L, _L)]
                    + mem_v[i, pl.ds(db * _L, _L)]
                    + tvec)

    start_in(0, nbuf0, tbuf0, sin0)
    start_in(1, nbuf1, tbuf1, sin1)
    wait_in(nbuf0, tbuf0, sin0)
    compute(0, nbuf0, tbuf0, obuf0)
    start_out(0, obuf0, sout0)
    start_in(2, nbuf0, tbuf0, sin0)
    wait_in(nbuf1, tbuf1, sin1)
    compute(1, nbuf1, tbuf1, obuf1)
    start_out(1, obuf1, sout1)
    start_in(3, nbuf1, tbuf1, sin1)

    def pair_body(cc, carry):
        c0 = cc * 2
        wait_in(nbuf0, tbuf0, sin0)
        wait_out(obuf0, sout0)
        compute(c0, nbuf0, tbuf0, obuf0)
        start_out(c0, obuf0, sout0)
        start_in(c0 + 2, nbuf0, tbuf0, sin0)
        wait_in(nbuf1, tbuf1, sin1)
        wait_out(obuf1, sout1)
        compute(c0 + 1, nbuf1, tbuf1, obuf1)
        start_out(c0 + 1, obuf1, sout1)
        start_in(c0 + 3, nbuf1, tbuf1, sin1)
        return carry

    lax.fori_loop(1, nchunks // 2 - 1, pair_body, 0)

    c0 = nchunks - 2
    wait_in(nbuf0, tbuf0, sin0)
    wait_out(obuf0, sout0)
    compute(c0, nbuf0, tbuf0, obuf0)
    start_out(c0, obuf0, sout0)
    wait_in(nbuf1, tbuf1, sin1)
    wait_out(obuf1, sout1)
    compute(c0 + 1, nbuf1, tbuf1, obuf1)
    start_out(c0 + 1, obuf1, sout1)
    wait_out(obuf0, sout0)
    wait_out(obuf1, sout1)


def kernel(x, mem):
    noise = jax.random.normal(jax.random.key(42), (_N, _D), jnp.float32) * 0.1
    t = pl.pallas_call(
        _scalar_stage,
        out_shape=jax.ShapeDtypeStruct((_M, _M), jnp.float32),
    )(x, x.T, mem, mem.T)
    tE = jnp.broadcast_to(t.reshape(_N, 1), (_N, _L))
    mesh = plsc.VectorSubcoreMesh(core_axis_name="c", subcore_axis_name="s")
    assemble = functools.partial(
        pl.kernel,
        mesh=mesh,
        out_type=jax.ShapeDtypeStruct((_NK, _D), jnp.float32),
        scratch_types=[
            pltpu.VMEM((_M, _D), jnp.float32),
            pltpu.VMEM((_CH, _D), jnp.float32),
            pltpu.VMEM((_CH, _D), jnp.float32),
            pltpu.VMEM((_CH, _D), jnp.float32),
            pltpu.VMEM((_CH, _D), jnp.float32),
            pltpu.VMEM((_CH, _L), jnp.float32),
            pltpu.VMEM((_CH, _L), jnp.float32),
            pltpu.SemaphoreType.DMA,
            pltpu.SemaphoreType.DMA,
            pltpu.SemaphoreType.DMA,
            pltpu.SemaphoreType.DMA,
        ],
        compiler_params=pltpu.CompilerParams(use_tc_tiling_on_sc=True),
    )(_sc_assemble)
    parts = []
    for k in range(_K):
        nz_k = lax.slice(noise, (k * _NK, 0), ((k + 1) * _NK, _D))
        tE_k = lax.slice(tE, (k * _NK, 0), ((k + 1) * _NK, _L))
        parts.append(assemble(nz_k, mem, tE_k))
    return jnp.concatenate(parts, axis=0)


# speedup vs baseline: 1.0901x; 1.0026x over previous
import functools

import jax
import jax.numpy as jnp
from jax import lax
from jax.experimental import pallas as pl
from jax.experimental.pallas import tpu as pltpu
from jax.experimental.pallas import tpu_sc as plsc

_M = 256
_D = 256
_N = _M * _M

_NC = 2
_NS = 16
_L = 16
_NW = _NC * _NS
_RPW = _N // _NW
_CH = 32
_CPA = _M // _CH


def _scalar_stage(x_ref, xT_ref, mem_ref, memT_ref, t_ref):
    x = x_ref[...]
    xT = xT_ref[...]
    mem = mem_ref[...]
    memT = memT_ref[...]
    f32 = jnp.float32
    rx = 1.0 / jnp.maximum(jnp.sqrt(jnp.sum(x * x, axis=1, keepdims=True)), 1e-8)
    xn_row = x * rx
    xn_col = xT * rx
    q1_row = jnp.sum(memT, axis=0, keepdims=True)
    q2_row = jnp.sum(memT * memT, axis=0, keepdims=True)
    q2_col = jnp.sum(mem * mem, axis=1, keepdims=True)
    rn_row = 1.0 / jnp.maximum(jnp.sqrt(q2_row), 1e-8)
    rn_col = 1.0 / jnp.maximum(jnp.sqrt(q2_col), 1e-8)
    mx_col = jnp.dot(mem, xn_col, preferred_element_type=f32)
    mx_row = jnp.dot(xn_row, memT, preferred_element_type=f32)
    sims_col = mx_col * rn_col
    sims_row = mx_row * rn_row
    act_col = (sims_col > 0.3).astype(f32)
    act_row = (sims_row > 0.3).astype(f32)
    raw = jnp.dot(mem, memT, preferred_element_type=f32)
    s = act_col * (raw * rn_col * rn_row) + sims_row
    v = xn_row + jnp.dot(act_row * rn_row, mem,
                         preferred_element_type=f32) * (1.0 / _M)
    sv = jnp.sum(v, axis=1, keepdims=True)
    dv_row = jnp.dot(v, memT, preferred_element_type=f32)
    den = jnp.maximum(jnp.sqrt(q2_row + 2.0 * s * q1_row + float(_D) * s * s),
                      1e-8)
    t_ref[...] = s + (dv_row + s * sv) / den


def _sc_assemble(noise_hbm, mem_hbm, tE_hbm, out_hbm, mem_v,
                 nbuf0, nbuf1, obuf0, obuf1, tbuf0, tbuf1,
                 sin0, sin1, sout0, sout1):
    wid = lax.axis_index("s") * _NC + lax.axis_index("c")
    pltpu.sync_copy(mem_hbm, mem_v)
    row0 = wid * _RPW
    nchunks = _RPW // _CH

    def start_in(c, nb, tb, sem):
        pltpu.async_copy(noise_hbm.at[pl.ds(row0 + c * _CH, _CH)], nb, sem)
        pltpu.async_copy(tE_hbm.at[pl.ds(row0 + c * _CH, _CH)], tb, sem)

    def wait_in(nb, tb, sem):
        pltpu.make_async_copy(noise_hbm.at[pl.ds(row0, _CH)], nb, sem).wait()
        pltpu.make_async_copy(tE_hbm.at[pl.ds(row0, _CH)], tb, sem).wait()

    def start_out(c, ob, sem):
        pltpu.async_copy(ob, out_hbm.at[pl.ds(row0 + c * _CH, _CH)], sem)

    def wait_out(ob, sem):
        pltpu.make_async_copy(ob, out_hbm.at[pl.ds(row0, _CH)], sem).wait()

    def compute(c, nb, tb, ob):
        i0 = (c % _CPA) * _CH

        @plsc.parallel_loop(0, _CH, unroll=8)
        def _row(rr):
            tvec = tb[rr, pl.ds(0, _L)]
            i = i0 + rr
            for db in range(_D // _L):
                ob[rr, pl.ds(db * _L, _L)] = (
                    nb[rr, pl.ds(db * _L, _L)]
                    + mem_v[i, pl.ds(db * _L, _L)]
                    + tvec)

    start_in(0, nbuf0, tbuf0, sin0)
    start_in(1, nbuf1, tbuf1, sin1)
    wait_in(nbuf0, tbuf0, sin0)
    compute(0, nbuf0, tbuf0, obuf0)
    start_out(0, obuf0, sout0)
    start_in(2, nbuf0, tbuf0, sin0)
    wait_in(nbuf1, tbuf1, sin1)
    compute(1, nbuf1, tbuf1, obuf1)
    start_out(1, obuf1, sout1)
    start_in(3, nbuf1, tbuf1, sin1)

    def pair_body(cc, carry):
        c0 = cc * 2
        wait_in(nbuf0, tbuf0, sin0)
        wait_out(obuf0, sout0)
        compute(c0, nbuf0, tbuf0, obuf0)
        start_out(c0, obuf0, sout0)
        start_in(c0 + 2, nbuf0, tbuf0, sin0)
        wait_in(nbuf1, tbuf1, sin1)
        wait_out(obuf1, sout1)
        compute(c0 + 1, nbuf1, tbuf1, obuf1)
        start_out(c0 + 1, obuf1, sout1)
        start_in(c0 + 3, nbuf1, tbuf1, sin1)
        return carry

    lax.fori_loop(1, nchunks // 2 - 1, pair_body, 0)

    c0 = nchunks - 2
    wait_in(nbuf0, tbuf0, sin0)
    wait_out(obuf0, sout0)
    compute(c0, nbuf0, tbuf0, obuf0)
    start_out(c0, obuf0, sout0)
    wait_in(nbuf1, tbuf1, sin1)
    wait_out(obuf1, sout1)
    compute(c0 + 1, nbuf1, tbuf1, obuf1)
    start_out(c0 + 1, obuf1, sout1)
    wait_out(obuf0, sout0)
    wait_out(obuf1, sout1)


def kernel(x, mem):
    noise = jax.random.normal(jax.random.key(42), (_N, _D), jnp.float32) * 0.1
    t = pl.pallas_call(
        _scalar_stage,
        out_shape=jax.ShapeDtypeStruct((_M, _M), jnp.float32),
    )(x, x.T, mem, mem.T)
    tE = jnp.broadcast_to(t.reshape(_N, 1), (_N, _L))
    mesh = plsc.VectorSubcoreMesh(core_axis_name="c", subcore_axis_name="s")
    assemble = functools.partial(
        pl.kernel,
        mesh=mesh,
        out_type=jax.ShapeDtypeStruct((_N, _D), jnp.float32),
        scratch_types=[
            pltpu.VMEM((_M, _D), jnp.float32),
            pltpu.VMEM((_CH, _D), jnp.float32),
            pltpu.VMEM((_CH, _D), jnp.float32),
            pltpu.VMEM((_CH, _D), jnp.float32),
            pltpu.VMEM((_CH, _D), jnp.float32),
            pltpu.VMEM((_CH, _L), jnp.float32),
            pltpu.VMEM((_CH, _L), jnp.float32),
            pltpu.SemaphoreType.DMA,
            pltpu.SemaphoreType.DMA,
            pltpu.SemaphoreType.DMA,
            pltpu.SemaphoreType.DMA,
        ],
        compiler_params=pltpu.CompilerParams(use_tc_tiling_on_sc=True),
    )(_sc_assemble)
    return assemble(noise, mem, tE)
